# Initial kernel scaffold; baseline (speedup 1.0000x reference)
import jax
import jax.numpy as jnp
from jax.experimental import pallas as pl


def kernel(data, edge_index, batch, W1a, b1a, W1b, b1b, W2a, b2a, W2b, b2b, W3a, b3a, W3b, b3b, g1, be1, g2, be2, g3, be3, Wlb, blb, Wlm, blm):
    raise NotImplementedError("write your pallas kernel here")


# speedup vs baseline: 9.0473x; 9.0473x over previous
import functools

import jax
import jax.numpy as jnp
from jax import lax
from jax.experimental import pallas as pl
from jax.experimental.pallas import tpu as pltpu
from jax.experimental.pallas import tpu_sc as plsc

_NS = 16
_NC = 2
_B = 512
_R = 128
_NR = _B // _R


@functools.lru_cache(maxsize=None)
def _make_agg(N, E_pad, H):
    HALF = N // 2
    ZR8 = ((HALF // _NS) + 7) // 8 * 8
    LAST = HALF - (_NS - 1) * ZR8
    assert LAST > 0 and LAST % 8 == 0 and HALF % 8 == 0
    CH = E_pad // _NS
    NBLK = CH // _B
    CHR = CH // _R
    ZC = 64
    mesh = plsc.VectorSubcoreMesh(core_axis_name="c", subcore_axis_name="s")

    def body(x_hbm, src_hbm, dst_hbm, out_hbm, src_v, dst_v, rows_v,
             zrows_v, acc_sh, gsem, ssem):
        c = lax.axis_index("c")
        s = lax.axis_index("s")
        cH = c * HALF
        zero16 = jnp.zeros((16,), jnp.float32)

        def _zrow(i, _):
            def _zcol(k, __):
                zrows_v[i, pl.ds(k * 16, 16)] = zero16
                return 0
            return lax.fori_loop(0, H // 16, _zcol, 0)
        lax.fori_loop(0, ZC, _zrow, 0)

        base = s * ZR8

        def _zero_stripe(nrows):
            def _zchunk(k, _):
                pltpu.sync_copy(zrows_v,
                                acc_sh.at[pl.ds(base + k * ZC, ZC)])
                return 0
            lax.fori_loop(0, nrows // ZC, _zchunk, 0)
            rem = nrows % ZC
            if rem:
                pltpu.sync_copy(
                    zrows_v.at[pl.ds(0, rem)],
                    acc_sh.at[pl.ds(base + (nrows // ZC) * ZC, rem)])

        @pl.when(s < _NS - 1)
        def _():
            _zero_stripe(ZR8)

        @pl.when(s == _NS - 1)
        def _():
            _zero_stripe(LAST)
        plsc.subcore_barrier()

        r0 = s * CHR

        def _blk(b, _):
            rbase = r0 + b * _NR
            pltpu.sync_copy(src_hbm.at[pl.ds(rbase, _NR)], src_v)
            gs = [pltpu.async_copy(x_hbm.at[src_v.at[j]],
                                   rows_v.at[pl.ds(j * _R, _R)], gsem)
                  for j in range(_NR)]
            pltpu.sync_copy(dst_hbm.at[pl.ds(rbase, _NR)], dst_v)

            def _fix(j, _):
                def _fix2(t, __):
                    v = dst_v[j, pl.ds(t * 16, 16)] - cH
                    vu = plsc.bitcast(v, jnp.uint32)
                    vu = jnp.minimum(vu, jnp.uint32(HALF))
                    dst_v[j, pl.ds(t * 16, 16)] = plsc.bitcast(vu, jnp.int32)
                    return 0
                return lax.fori_loop(0, _R // 16, _fix2, 0)
            lax.fori_loop(0, _NR, _fix, 0)

            for h in gs:
                h.wait()
            sc = [pltpu.async_copy(rows_v.at[pl.ds(j * _R, _R)],
                                   acc_sh.at[dst_v.at[j]], ssem, add=True)
                  for j in range(_NR)]
            for h in sc:
                h.wait()
            return 0
        lax.fori_loop(0, NBLK, _blk, 0)
        plsc.subcore_barrier()

        @pl.when(s < _NS - 1)
        def _():
            pltpu.sync_copy(acc_sh.at[pl.ds(base, ZR8)],
                            out_hbm.at[pl.ds(cH + base, ZR8)])

        @pl.when(s == _NS - 1)
        def _():
            pltpu.sync_copy(acc_sh.at[pl.ds(base, LAST)],
                            out_hbm.at[pl.ds(cH + base, LAST)])

    return pl.kernel(
        body,
        out_type=jax.ShapeDtypeStruct((N, H), jnp.float32),
        mesh=mesh,
        compiler_params=pltpu.CompilerParams(use_tc_tiling_on_sc=False),
        scratch_types=[
            pltpu.VMEM((_NR, _R), jnp.int32),
            pltpu.VMEM((_NR, _R), jnp.int32),
            pltpu.VMEM((_B, H), jnp.float32),
            pltpu.VMEM((ZC, H), jnp.float32),
            pltpu.VMEM_SHARED((HALF + 8, H), jnp.float32),
            pltpu.SemaphoreType.DMA,
            pltpu.SemaphoreType.DMA,
        ],
    )


def _premix(data8, W8):
    Np = data8.shape[0]
    BLK = 2000
    Dp = data8.shape[1]
    H = W8.shape[1]

    def body(x_ref, w_ref, o_ref):
        o_ref[...] = jnp.dot(x_ref[...], w_ref[...],
                             preferred_element_type=jnp.float32, precision=lax.Precision.HIGHEST)

    return pl.pallas_call(
        body,
        grid=(Np // BLK,),
        in_specs=[pl.BlockSpec((BLK, Dp), lambda i: (i, 0)),
                  pl.BlockSpec((Dp, H), lambda i: (0, 0))],
        out_specs=pl.BlockSpec((BLK, H), lambda i: (i, 0)),
        out_shape=jax.ShapeDtypeStruct((Np, H), jnp.float32),
    )(data8, W8)


def _gin_layer(x, u, Wa, Wb, BP):
    Np, H = x.shape
    BLK = 2000

    def body(x_ref, u_ref, wa_ref, wb_ref, bp_ref, c_ref, s_ref):
        z = x_ref[...] + u_ref[...]
        h = jnp.maximum(jnp.dot(z, wa_ref[...],
                                preferred_element_type=jnp.float32, precision=lax.Precision.HIGHEST)
                        + bp_ref[0:1, :], 0.0)
        p = jnp.dot(h, wb_ref[...], preferred_element_type=jnp.float32, precision=lax.Precision.HIGHEST) \
            + bp_ref[1:2, :]
        cel = jnp.where(p > 0.0, p, jnp.exp(jnp.minimum(p, 0.0)) - 1.0)
        c_ref[...] = cel
        ps = jnp.sum(cel, axis=0, keepdims=True)
        pq = jnp.sum(cel * cel, axis=0, keepdims=True)
        blksums = jnp.concatenate(
            [ps, pq, jnp.zeros((6, H), jnp.float32)], axis=0)

        @pl.when(pl.program_id(0) == 0)
        def _():
            s_ref[...] = jnp.zeros_like(s_ref)
        s_ref[...] += blksums

    return pl.pallas_call(
        body,
        grid=(Np // BLK,),
        in_specs=[
            pl.BlockSpec((BLK, H), lambda i: (i, 0)),
            pl.BlockSpec((BLK, H), lambda i: (i, 0)),
            pl.BlockSpec((H, H), lambda i: (0, 0)),
            pl.BlockSpec((H, H), lambda i: (0, 0)),
            pl.BlockSpec((8, H), lambda i: (0, 0)),
        ],
        out_specs=[pl.BlockSpec((BLK, H), lambda i: (i, 0)),
                   pl.BlockSpec((8, H), lambda i: (0, 0))],
        out_shape=[jax.ShapeDtypeStruct((Np, H), jnp.float32),
                   jax.ShapeDtypeStruct((8, H), jnp.float32)],
    )(x, u, Wa, Wb, BP)


def _norm(c, ST):
    Np, H = c.shape
    BLK = 2000

    def body(c_ref, st_ref, o_ref):
        o_ref[...] = c_ref[...] * st_ref[0:1, :] + st_ref[1:2, :]

    return pl.pallas_call(
        body,
        grid=(Np // BLK,),
        in_specs=[pl.BlockSpec((BLK, H), lambda i: (i, 0)),
                  pl.BlockSpec((8, H), lambda i: (0, 0))],
        out_specs=pl.BlockSpec((BLK, H), lambda i: (i, 0)),
        out_shape=jax.ShapeDtypeStruct((Np, H), jnp.float32),
    )(c, ST)


def _segmax_head(c3, ST, starts, Wlb, Wlm, HB, G):
    Np, H = c3.shape
    H2 = Wlb.shape[1]
    CHK = 128
    NEG = float("-inf")

    def body(st_ref, c_ref, stv_ref, wlb_ref, wlm_ref, hb_ref, o_ref,
             emb_ref):
        sH = stv_ref[0:1, :]
        tH = stv_ref[1:2, :]
        iota = lax.broadcasted_iota(jnp.int32, (CHK, 1), 0)

        def seg(g, _):
            st = st_ref[g]
            en = st_ref[g + 1]
            nit = (en - st + CHK - 1) // CHK

            def it(i, acc):
                off = jnp.minimum(st + i * CHK, Np - CHK)
                rows = c_ref[pl.ds(off, CHK), :]
                idx = off + iota
                valid = (idx >= st) & (idx < en)
                val = jnp.where(valid, rows * sH + tH, NEG)
                return jnp.maximum(acc, jnp.max(val, axis=0, keepdims=True))

            acc = lax.fori_loop(0, nit, it,
                                jnp.full((1, H), NEG, jnp.float32))
            emb_ref[pl.ds(g, 1), :] = acc
            return 0

        lax.fori_loop(0, G, seg, 0)
        e = emb_ref[...]
        hh = jnp.dot(e, wlb_ref[...], preferred_element_type=jnp.float32, precision=lax.Precision.HIGHEST) \
            + hb_ref[0:1, :]
        cel = jnp.where(hh > 0.0, hh, jnp.exp(jnp.minimum(hh, 0.0)) - 1.0)
        lo = jnp.dot(cel, wlm_ref[...],
                     preferred_element_type=jnp.float32, precision=lax.Precision.HIGHEST) + hb_ref[1:2, 0:1]
        o_ref[...] = 1.0 / (1.0 + jnp.exp(-lo))

    return pl.pallas_call(
        body,
        grid=(1,),
        in_specs=[
            pl.BlockSpec(memory_space=pltpu.SMEM),
            pl.BlockSpec((Np, H), lambda i: (0, 0)),
            pl.BlockSpec((8, H), lambda i: (0, 0)),
            pl.BlockSpec((H, H2), lambda i: (0, 0)),
            pl.BlockSpec((H2, 1), lambda i: (0, 0)),
            pl.BlockSpec((8, H2), lambda i: (0, 0)),
        ],
        out_specs=pl.BlockSpec((G, 1), lambda i: (0, 0)),
        out_shape=jax.ShapeDtypeStruct((G, 1), jnp.float32),
        scratch_shapes=[pltpu.VMEM((G, H), jnp.float32)],
    )(starts, c3, ST, Wlb, Wlm, HB)


def _pack2(a, b):
    H = a.shape[0]
    return jnp.concatenate(
        [a[None, :], b[None, :], jnp.zeros((6, H), jnp.float32)], axis=0)


def _stats(sums, g, be, N):
    m = sums[0] / N
    ex2 = sums[1] / N
    v = ex2 - m * m
    s = g * lax.rsqrt(v + 1e-5)
    t = be - m * s
    return _pack2(s, t)


def kernel(data, edge_index, batch, W1a, b1a, W1b, b1b, W2a, b2a, W2b, b2b,
           W3a, b3a, W3b, b3b, g1, be1, g2, be2, g3, be3, Wlb, blb, Wlm,
           blm):
    N, D = data.shape
    H = W1a.shape[1]
    E = edge_index.shape[1]
    G = 1024

    gran = _NS * _B
    E_pad = -(-E // gran) * gran
    pad = E_pad - E
    src_p = jnp.concatenate(
        [edge_index[0], jnp.zeros((pad,), jnp.int32)]).reshape(
            E_pad // _R, _R)
    dst_p = jnp.concatenate(
        [edge_index[1], jnp.full((pad,), N, jnp.int32)]).reshape(
            E_pad // _R, _R)
    agg = _make_agg(N, E_pad, H)

    data8 = jnp.pad(data, ((0, 0), (0, 8 - D)))
    W1a8 = jnp.pad(W1a, ((0, 8 - D), (0, 0)))
    y0 = _premix(data8, W1a8)
    eye = jnp.eye(H, dtype=jnp.float32)

    u0 = agg(y0, src_p, dst_p)
    c1, sums1 = _gin_layer(y0, u0, eye, W1b, _pack2(b1a, b1b))
    x1 = _norm(c1, _stats(sums1, g1, be1, N))

    u1 = agg(x1, src_p, dst_p)
    c2, sums2 = _gin_layer(x1, u1, W2a, W2b, _pack2(b2a, b2b))
    x2 = _norm(c2, _stats(sums2, g2, be2, N))

    u2 = agg(x2, src_p, dst_p)
    c3, sums3 = _gin_layer(x2, u2, W3a, W3b, _pack2(b3a, b3b))
    st3 = _stats(sums3, g3, be3, N)

    starts = jnp.searchsorted(
        batch, jnp.arange(G + 1, dtype=jnp.int32)).astype(jnp.int32)
    HB = _pack2(blb, jnp.pad(blm, (0, Wlb.shape[1] - 1)))
    return _segmax_head(c3, st3, starts, Wlb, Wlm, HB, G)
